# baseline (speedup 1.0000x reference)
import functools

import jax
import jax.numpy as jnp
from jax import lax
from jax.experimental import pallas as pl
from jax.experimental.pallas import tpu as pltpu
from jax.experimental.pallas import tpu_sc as plsc

B = 16384
D = 128
NW = 32
PHASES = (12288, 4096)
CHUNK = 64
NGATHER = 7
NOUT = 5
BM = 512


@functools.cache
def _gather_sc(offset, rows):
    bpw = rows // NW
    nchunk = bpw // CHUNK
    mesh = plsc.VectorSubcoreMesh(core_axis_name="c", subcore_axis_name="s")
    scratch = (
        [pltpu.VMEM((bpw,), jnp.int32) for _ in range(3)]
        + [pltpu.VMEM((CHUNK, D), jnp.float32) for _ in range(2 * NGATHER)]
        + [pltpu.SemaphoreType.DMA for _ in range(2 * NGATHER)]
    )

    @functools.partial(
        pl.kernel,
        mesh=mesh,
        out_type=jax.ShapeDtypeStruct((NOUT, rows, D), jnp.float32),
        scratch_types=scratch,
    )
    def gather(head_hbm, tail_hbm, rel_hbm, rg_hbm, ap_hbm, gn_hbm, rem_hbm,
               out_hbm, *sc):
        idx = sc[0:3]
        bufs = [sc[3 + s * NGATHER:3 + (s + 1) * NGATHER] for s in (0, 1)]
        sems = [sc[17 + s * NGATHER:17 + (s + 1) * NGATHER] for s in (0, 1)]
        wid = lax.axis_index("s") * 2 + lax.axis_index("c")
        base = wid * bpw
        src_base = offset + wid * bpw
        for src, dst in zip((head_hbm, tail_hbm, rel_hbm), idx):
            pltpu.sync_copy(src.at[pl.ds(src_base, bpw)], dst)
        jobs = ((rg_hbm, 0), (rg_hbm, 1), (ap_hbm, 0), (ap_hbm, 1),
                (gn_hbm, 0), (rem_hbm, 2), (gn_hbm, 1))
        gh = [[None] * NGATHER for _ in (0, 1)]
        wh = [[None] * NOUT for _ in (0, 1)]

        def fire_gathers(j):
            s = j % 2
            for k, (tbl, which) in enumerate(jobs):
                if k < NOUT and wh[s][k] is not None:
                    wh[s][k].wait()
                gh[s][k] = pltpu.async_copy(
                    tbl.at[idx[which].at[pl.ds(j * CHUNK, CHUNK)]],
                    bufs[s][k], sems[s][k])

        def combine(s):
            a, bb, cc = bufs[s][4], bufs[s][5], bufs[s][6]

            def body(r, _):
                for c in range(D // 16):
                    sl = pl.ds(c * 16, 16)
                    a[r, sl] = a[r, sl] + bb[r, sl] - cc[r, sl]
                return 0

            lax.fori_loop(0, CHUNK, body, 0)

        fire_gathers(0)
        for j in range(nchunk):
            if j + 1 < nchunk:
                fire_gathers(j + 1)
            s = j % 2
            row = pl.ds(base + j * CHUNK, CHUNK)
            for k in range(4):
                gh[s][k].wait()
                wh[s][k] = pltpu.async_copy(bufs[s][k], out_hbm.at[k, row],
                                            sems[s][k])
            for k in range(4, NGATHER):
                gh[s][k].wait()
            combine(s)
            wh[s][4] = pltpu.async_copy(bufs[s][4], out_hbm.at[4, row],
                                        sems[s][4])
        for s in (0, 1):
            for k in range(NOUT):
                if wh[s][k] is not None:
                    wh[s][k].wait()

    return gather


def _mlp_tc(o_ref, w1t_ref, b1_ref, w2c_ref, b2c_ref, out_ref):
    f32 = jnp.float32
    acc = jnp.dot(o_ref[0], w1t_ref[0:128], preferred_element_type=f32)
    acc += jnp.dot(o_ref[1], w1t_ref[128:256], preferred_element_type=f32)
    acc += jnp.dot(o_ref[2], w1t_ref[256:384], preferred_element_type=f32)
    acc += jnp.dot(o_ref[3], w1t_ref[384:512], preferred_element_type=f32)
    acc += jnp.dot(o_ref[4], w1t_ref[512:640], preferred_element_type=f32)
    hid = jnp.maximum(acc + b1_ref[...], 0.0)
    logits = jnp.dot(hid, w2c_ref[...], preferred_element_type=f32) + b2c_ref[...]
    out_ref[...] = 1.0 / (1.0 + jnp.exp(-logits))


def _mlp_carry_tc(o_ref, w1t_ref, b1_ref, w2c_ref, b2c_ref, carry_ref, out_ref):
    del carry_ref
    _mlp_tc(o_ref, w1t_ref, b1_ref, w2c_ref, b2c_ref, out_ref)


def _common_specs():
    return [
        pl.BlockSpec((NOUT, BM, D), lambda i: (0, i, 0)),
        pl.BlockSpec((5 * D, D), lambda i: (0, 0)),
        pl.BlockSpec((1, D), lambda i: (0, 0)),
        pl.BlockSpec((D, 2), lambda i: (0, 0)),
        pl.BlockSpec((1, 2), lambda i: (0, 0)),
    ]


def _mlp_single(gathered, w1t, b1r, w2c, b2c):
    return pl.pallas_call(
        _mlp_tc,
        grid=(B // BM,),
        in_specs=_common_specs(),
        out_specs=pl.BlockSpec((BM, 2), lambda i: (i, 0)),
        out_shape=jax.ShapeDtypeStruct((B, 2), jnp.float32),
    )(gathered, w1t, b1r, w2c, b2c)


def _mlp_phase(offset, rows, gathered, w1t, b1r, w2c, b2c, carry):
    blk_off = offset // BM
    return pl.pallas_call(
        _mlp_carry_tc,
        grid=(rows // BM,),
        in_specs=_common_specs() + [pl.BlockSpec(memory_space=pl.ANY)],
        out_specs=pl.BlockSpec((BM, 2), lambda i: (blk_off + i, 0)),
        out_shape=jax.ShapeDtypeStruct((B, 2), jnp.float32),
        input_output_aliases={5: 0},
    )(gathered, w1t, b1r, w2c, b2c, carry)


def kernel(head, rel, tail, rg_feature, ap_feature, gn_feature, rel_emb,
           W1, b1, W2, b2):
    head = head.astype(jnp.int32)
    tail = tail.astype(jnp.int32)
    rel = rel.astype(jnp.int32)

    w1t = W1.T
    b1r = b1.reshape(1, D)
    wdiff = W2[0] - W2[1]
    w2c = jnp.stack([wdiff, -wdiff], axis=1)
    bdiff = b2[0] - b2[1]
    b2c = jnp.stack([bdiff, -bdiff]).reshape(1, 2)

    if len(PHASES) == 1:
        g = _gather_sc(0, B)(head, tail, rel, rg_feature, ap_feature,
                             gn_feature, rel_emb)
        return _mlp_single(g, w1t, b1r, w2c, b2c)

    out = jnp.zeros((B, 2), jnp.float32)
    offset = 0
    for rows in PHASES:
        g = _gather_sc(offset, rows)(head, tail, rel, rg_feature, ap_feature,
                                     gn_feature, rel_emb)
        out = _mlp_phase(offset, rows, g, w1t, b1r, w2c, b2c, out)
        offset += rows
    return out


# speedup vs baseline: 1.0572x; 1.0572x over previous
import functools

import jax
import jax.numpy as jnp
from jax import lax
from jax.experimental import pallas as pl
from jax.experimental.pallas import tpu as pltpu
from jax.experimental.pallas import tpu_sc as plsc

B = 16384
D = 128
NW = 32
PHASES = (8192, 8192)
CHUNK = 64
NGATHER = 7
NOUT = 5
BM = 512


@functools.cache
def _gather_sc(offset, rows):
    bpw = rows // NW
    nchunk = bpw // CHUNK
    mesh = plsc.VectorSubcoreMesh(core_axis_name="c", subcore_axis_name="s")
    scratch = (
        [pltpu.VMEM((bpw,), jnp.int32) for _ in range(3)]
        + [pltpu.VMEM((CHUNK, D), jnp.float32) for _ in range(2 * NGATHER)]
        + [pltpu.SemaphoreType.DMA for _ in range(2 * NGATHER)]
    )

    @functools.partial(
        pl.kernel,
        mesh=mesh,
        out_type=jax.ShapeDtypeStruct((NOUT, rows, D), jnp.float32),
        scratch_types=scratch,
    )
    def gather(head_hbm, tail_hbm, rel_hbm, rg_hbm, ap_hbm, gn_hbm, rem_hbm,
               out_hbm, *sc):
        idx = sc[0:3]
        bufs = [sc[3 + s * NGATHER:3 + (s + 1) * NGATHER] for s in (0, 1)]
        sems = [sc[17 + s * NGATHER:17 + (s + 1) * NGATHER] for s in (0, 1)]
        wid = lax.axis_index("s") * 2 + lax.axis_index("c")
        base = wid * bpw
        src_base = offset + wid * bpw
        for src, dst in zip((head_hbm, tail_hbm, rel_hbm), idx):
            pltpu.sync_copy(src.at[pl.ds(src_base, bpw)], dst)
        jobs = ((rg_hbm, 0), (rg_hbm, 1), (ap_hbm, 0), (ap_hbm, 1),
                (gn_hbm, 0), (rem_hbm, 2), (gn_hbm, 1))
        gh = [[None] * NGATHER for _ in (0, 1)]
        wh = [[None] * NOUT for _ in (0, 1)]

        def fire_gathers(j):
            s = j % 2
            for k, (tbl, which) in enumerate(jobs):
                if k < NOUT and wh[s][k] is not None:
                    wh[s][k].wait()
                gh[s][k] = pltpu.async_copy(
                    tbl.at[idx[which].at[pl.ds(j * CHUNK, CHUNK)]],
                    bufs[s][k], sems[s][k])

        def combine(s):
            a, bb, cc = bufs[s][4], bufs[s][5], bufs[s][6]

            def body(r, _):
                for c in range(D // 16):
                    sl = pl.ds(c * 16, 16)
                    a[r, sl] = a[r, sl] + bb[r, sl] - cc[r, sl]
                return 0

            lax.fori_loop(0, CHUNK, body, 0)

        fire_gathers(0)
        for j in range(nchunk):
            if j + 1 < nchunk:
                fire_gathers(j + 1)
            s = j % 2
            row = pl.ds(base + j * CHUNK, CHUNK)
            for k in range(4):
                gh[s][k].wait()
                wh[s][k] = pltpu.async_copy(bufs[s][k], out_hbm.at[k, row],
                                            sems[s][k])
            for k in range(4, NGATHER):
                gh[s][k].wait()
            combine(s)
            wh[s][4] = pltpu.async_copy(bufs[s][4], out_hbm.at[4, row],
                                        sems[s][4])
        for s in (0, 1):
            for k in range(NOUT):
                if wh[s][k] is not None:
                    wh[s][k].wait()

    return gather


def _mlp_tc(o_ref, w1t_ref, b1_ref, w2c_ref, b2c_ref, out_ref):
    f32 = jnp.float32
    acc = jnp.dot(o_ref[0], w1t_ref[0:128], preferred_element_type=f32)
    acc += jnp.dot(o_ref[1], w1t_ref[128:256], preferred_element_type=f32)
    acc += jnp.dot(o_ref[2], w1t_ref[256:384], preferred_element_type=f32)
    acc += jnp.dot(o_ref[3], w1t_ref[384:512], preferred_element_type=f32)
    acc += jnp.dot(o_ref[4], w1t_ref[512:640], preferred_element_type=f32)
    hid = jnp.maximum(acc + b1_ref[...], 0.0)
    logits = jnp.dot(hid, w2c_ref[...], preferred_element_type=f32) + b2c_ref[...]
    out_ref[...] = 1.0 / (1.0 + jnp.exp(-logits))


def _mlp_carry_tc(o_ref, w1t_ref, b1_ref, w2c_ref, b2c_ref, carry_ref, out_ref):
    del carry_ref
    _mlp_tc(o_ref, w1t_ref, b1_ref, w2c_ref, b2c_ref, out_ref)


def _common_specs():
    return [
        pl.BlockSpec((NOUT, BM, D), lambda i: (0, i, 0)),
        pl.BlockSpec((5 * D, D), lambda i: (0, 0)),
        pl.BlockSpec((1, D), lambda i: (0, 0)),
        pl.BlockSpec((D, 2), lambda i: (0, 0)),
        pl.BlockSpec((1, 2), lambda i: (0, 0)),
    ]


def _mlp_first(rows, gathered, w1t, b1r, w2c, b2c):
    return pl.pallas_call(
        _mlp_tc,
        grid=(rows // BM,),
        in_specs=_common_specs(),
        out_specs=pl.BlockSpec((BM, 2), lambda i: (i, 0)),
        out_shape=jax.ShapeDtypeStruct((B, 2), jnp.float32),
    )(gathered, w1t, b1r, w2c, b2c)


def _mlp_phase(offset, rows, gathered, w1t, b1r, w2c, b2c, carry):
    blk_off = offset // BM
    return pl.pallas_call(
        _mlp_carry_tc,
        grid=(rows // BM,),
        in_specs=_common_specs() + [pl.BlockSpec(memory_space=pl.ANY)],
        out_specs=pl.BlockSpec((BM, 2), lambda i: (blk_off + i, 0)),
        out_shape=jax.ShapeDtypeStruct((B, 2), jnp.float32),
        input_output_aliases={5: 0},
    )(gathered, w1t, b1r, w2c, b2c, carry)


def kernel(head, rel, tail, rg_feature, ap_feature, gn_feature, rel_emb,
           W1, b1, W2, b2):
    head = head.astype(jnp.int32)
    tail = tail.astype(jnp.int32)
    rel = rel.astype(jnp.int32)

    w1t = W1.T
    b1r = b1.reshape(1, D)
    wdiff = W2[0] - W2[1]
    w2c = jnp.stack([wdiff, -wdiff], axis=1)
    bdiff = b2[0] - b2[1]
    b2c = jnp.stack([bdiff, -bdiff]).reshape(1, 2)

    out = None
    offset = 0
    for rows in PHASES:
        g = _gather_sc(offset, rows)(head, tail, rel, rg_feature, ap_feature,
                                     gn_feature, rel_emb)
        if out is None:
            out = _mlp_first(rows, g, w1t, b1r, w2c, b2c)
        else:
            out = _mlp_phase(offset, rows, g, w1t, b1r, w2c, b2c, out)
        offset += rows
    return out
